# Initial kernel scaffold; baseline (speedup 1.0000x reference)
import jax
import jax.numpy as jnp
from jax.experimental import pallas as pl


def kernel(x_nchw, w_hwio, bias):
    raise NotImplementedError("write your pallas kernel here")


# speedup vs baseline: 2.5523x; 2.5523x over previous
import jax
import jax.numpy as jnp
from jax.experimental import pallas as pl
from jax.experimental.pallas import tpu as pltpu


def _conv3x3_kernel(x_ref, halo_ref, w_ref, b_ref, o_ref):
    th = x_ref.shape[1]
    W = x_ref.shape[2]
    C = x_ref.shape[3]
    Cout = w_ref.shape[3]
    S = th * W

    xw = jnp.concatenate([halo_ref[0, 0:1], x_ref[0], halo_ref[0, 1:2]],
                         axis=0)

    z = jnp.zeros((th + 2, 1, C), xw.dtype)
    lft = jnp.concatenate([z, xw[:, :W - 1, :]], axis=1)
    rgt = jnp.concatenate([xw[:, 1:, :], z], axis=1)

    acc = jnp.broadcast_to(b_ref[...], (S, Cout)).astype(jnp.float32)
    for kh in range(3):
        for kw, buf in ((0, lft), (1, xw), (2, rgt)):
            patch = buf[kh:kh + th].reshape(S, C)
            acc = acc + jnp.dot(patch, w_ref[kh, kw],
                                preferred_element_type=jnp.float32)

    acc = jnp.maximum(acc, 0.0)
    o_ref[0] = acc.reshape(th, W, Cout)


def kernel(x_nchw, w_hwio, bias):
    N, Cin, H, W = x_nchw.shape
    kH, kW, _, Cout = w_hwio.shape
    assert (kH, kW) == (3, 3)

    tile_h = next(t for t in (32, 16, 8, 4, 2, 1) if H % t == 0)
    nH = H // tile_h

    x_nhwc = jnp.transpose(x_nchw, (0, 2, 3, 1)).astype(jnp.bfloat16)

    zrow = jnp.zeros((N, 1, W, Cin), jnp.bfloat16)
    top = jnp.concatenate([zrow, x_nhwc[:, tile_h - 1:H - 1:tile_h]], axis=1)
    bot = jnp.concatenate([x_nhwc[:, tile_h:H:tile_h], zrow], axis=1)
    halo = jnp.stack([top, bot], axis=2).reshape(N, 2 * nH, W, Cin)

    w_b = w_hwio.astype(jnp.bfloat16)
    b2 = bias.reshape(1, Cout).astype(jnp.float32)

    out_nhwc = pl.pallas_call(
        _conv3x3_kernel,
        out_shape=jax.ShapeDtypeStruct((N, H, W, Cout), jnp.float32),
        grid=(N, nH),
        in_specs=[
            pl.BlockSpec((1, tile_h, W, Cin), lambda n, h: (n, h, 0, 0)),
            pl.BlockSpec((1, 2, W, Cin), lambda n, h: (n, h, 0, 0)),
            pl.BlockSpec((3, 3, Cin, Cout), lambda n, h: (0, 0, 0, 0)),
            pl.BlockSpec((1, Cout), lambda n, h: (0, 0)),
        ],
        out_specs=pl.BlockSpec((1, tile_h, W, Cout), lambda n, h: (n, h, 0, 0)),
        compiler_params=pltpu.CompilerParams(
            dimension_semantics=("parallel", "parallel")),
    )(x_nhwc, halo, w_b, b2)

    return jnp.transpose(out_nhwc, (0, 3, 1, 2))


# speedup vs baseline: 2.5711x; 1.0074x over previous
import jax
import jax.numpy as jnp
from jax.experimental import pallas as pl
from jax.experimental.pallas import tpu as pltpu


def _conv3x3_kernel(x_ref, h_ref, ml_ref, mr_ref, w_ref, b_ref, o_ref,
                    pk_ref):
    C = x_ref.shape[1]
    SW = x_ref.shape[2]
    W = h_ref.shape[2] // 2
    Cout = o_ref.shape[1]

    hb = h_ref[0].astype(jnp.bfloat16)
    pk_ref[C:2 * C, 0:W] = hb[:, :W]
    pk_ref[C:2 * C, W:W + SW] = x_ref[0].astype(jnp.bfloat16)
    pk_ref[C:2 * C, W + SW:] = hb[:, W:]

    cen = pk_ref[C:2 * C, :]
    z1 = jnp.zeros((C, 1), jnp.bfloat16)
    pk_ref[0:C, :] = jnp.concatenate([z1, cen[:, :-1]], axis=1) * ml_ref[...]
    pk_ref[2 * C:3 * C, :] = (jnp.concatenate([cen[:, 1:], z1], axis=1)
                              * mr_ref[...])

    acc = jnp.broadcast_to(b_ref[...], (Cout, SW)).astype(jnp.float32)
    for kh in range(3):
        acc = acc + jnp.dot(w_ref[kh], pk_ref[:, kh * W:kh * W + SW],
                            preferred_element_type=jnp.float32)

    o_ref[0] = jnp.maximum(acc, 0.0)


def kernel(x_nchw, w_hwio, bias):
    N, Cin, H, W = x_nchw.shape
    kH, kW, _, Cout = w_hwio.shape
    assert (kH, kW) == (3, 3)

    tile_h = next(t for t in (32, 16, 8, 4, 2, 1) if H % t == 0)
    nH = H // tile_h
    SW = tile_h * W
    XW = SW + 2 * W

    x_flat = x_nchw.reshape(N, Cin, H * W)

    zrow = jnp.zeros((N, Cin, 1, W), x_nchw.dtype)
    top = jnp.concatenate([zrow, x_nchw[:, :, tile_h - 1:H - 1:tile_h]],
                          axis=2)
    bot = jnp.concatenate([x_nchw[:, :, tile_h:H:tile_h], zrow], axis=2)
    halo = jnp.stack([top, bot], axis=3).reshape(N, Cin, 2 * nH * W)

    lane = jnp.arange(XW, dtype=jnp.int32) % W
    ml = (lane != 0).astype(jnp.bfloat16).reshape(1, XW)
    mr = (lane != W - 1).astype(jnp.bfloat16).reshape(1, XW)

    w_k = jnp.transpose(w_hwio, (0, 3, 1, 2)).reshape(kH, Cout, kW * Cin)
    w_k = w_k.astype(jnp.bfloat16)
    b2 = bias.reshape(Cout, 1).astype(jnp.float32)

    out_flat = pl.pallas_call(
        _conv3x3_kernel,
        out_shape=jax.ShapeDtypeStruct((N, Cout, H * W), jnp.float32),
        grid_spec=pltpu.PrefetchScalarGridSpec(
            num_scalar_prefetch=0,
            grid=(N, nH),
            in_specs=[
                pl.BlockSpec((1, Cin, SW), lambda n, h: (n, 0, h)),
                pl.BlockSpec((1, Cin, 2 * W), lambda n, h: (n, 0, h)),
                pl.BlockSpec((1, XW), lambda n, h: (0, 0)),
                pl.BlockSpec((1, XW), lambda n, h: (0, 0)),
                pl.BlockSpec((kH, Cout, kW * Cin), lambda n, h: (0, 0, 0)),
                pl.BlockSpec((Cout, 1), lambda n, h: (0, 0)),
            ],
            out_specs=pl.BlockSpec((1, Cout, SW), lambda n, h: (n, 0, h)),
            scratch_shapes=[pltpu.VMEM((kW * Cin, XW), jnp.bfloat16)],
        ),
        compiler_params=pltpu.CompilerParams(
            dimension_semantics=("parallel", "parallel")),
    )(x_flat, halo, ml, mr, w_k, b2)

    return out_flat.reshape(N, Cout, H, W)


# speedup vs baseline: 4.4091x; 1.7148x over previous
import jax
import jax.numpy as jnp
from jax.experimental import pallas as pl
from jax.experimental.pallas import tpu as pltpu


def _conv3x3_kernel(x_ref, h_ref, ml_ref, mr_ref, w_ref, b_ref, o_ref,
                    pk_ref):
    C = x_ref.shape[1]
    th = x_ref.shape[2]
    W = x_ref.shape[3]
    SW = th * W
    Cout = o_ref.shape[1]

    hb = h_ref[0, :, 0].astype(jnp.bfloat16)
    xb = x_ref[0].astype(jnp.bfloat16).reshape(C, SW)
    pk_ref[C:2 * C, 0:W] = hb[:, 0, :]
    pk_ref[C:2 * C, W:W + SW] = xb
    pk_ref[C:2 * C, W + SW:] = hb[:, 1, :]

    cen = pk_ref[C:2 * C, :]
    z1 = jnp.zeros((C, 1), jnp.bfloat16)
    pk_ref[0:C, :] = jnp.concatenate([z1, cen[:, :-1]], axis=1) * ml_ref[...]
    pk_ref[2 * C:3 * C, :] = (jnp.concatenate([cen[:, 1:], z1], axis=1)
                              * mr_ref[...])

    acc = jnp.broadcast_to(b_ref[...], (Cout, SW)).astype(jnp.float32)
    for kh in range(3):
        acc = acc + jnp.dot(w_ref[kh], pk_ref[:, kh * W:kh * W + SW],
                            preferred_element_type=jnp.float32)

    o_ref[0] = jnp.maximum(acc, 0.0).reshape(Cout, th, W)


def kernel(x_nchw, w_hwio, bias):
    N, Cin, H, W = x_nchw.shape
    kH, kW, _, Cout = w_hwio.shape
    assert (kH, kW) == (3, 3)

    tile_h = next(t for t in (32, 16, 8, 4, 2, 1) if H % t == 0)
    nH = H // tile_h
    SW = tile_h * W
    XW = SW + 2 * W

    zrow = jnp.zeros((N, Cin, 1, W), x_nchw.dtype)
    top = jnp.concatenate([zrow, x_nchw[:, :, tile_h - 1:H - 1:tile_h]],
                          axis=2)
    bot = jnp.concatenate([x_nchw[:, :, tile_h:H:tile_h], zrow], axis=2)
    halo = jnp.stack([top, bot], axis=3)

    lane = jnp.arange(XW, dtype=jnp.int32) % W
    ml = (lane != 0).astype(jnp.bfloat16).reshape(1, XW)
    mr = (lane != W - 1).astype(jnp.bfloat16).reshape(1, XW)

    w_k = jnp.transpose(w_hwio, (0, 3, 1, 2)).reshape(kH, Cout, kW * Cin)
    w_k = w_k.astype(jnp.bfloat16)
    b2 = bias.reshape(Cout, 1).astype(jnp.float32)

    out = pl.pallas_call(
        _conv3x3_kernel,
        out_shape=jax.ShapeDtypeStruct((N, Cout, H, W), jnp.float32),
        grid_spec=pltpu.PrefetchScalarGridSpec(
            num_scalar_prefetch=0,
            grid=(N, nH),
            in_specs=[
                pl.BlockSpec((1, Cin, tile_h, W), lambda n, h: (n, 0, h, 0)),
                pl.BlockSpec((1, Cin, 1, 2, W), lambda n, h: (n, 0, h, 0, 0)),
                pl.BlockSpec((1, XW), lambda n, h: (0, 0)),
                pl.BlockSpec((1, XW), lambda n, h: (0, 0)),
                pl.BlockSpec((kH, Cout, kW * Cin), lambda n, h: (0, 0, 0)),
                pl.BlockSpec((Cout, 1), lambda n, h: (0, 0)),
            ],
            out_specs=pl.BlockSpec((1, Cout, tile_h, W),
                                   lambda n, h: (n, 0, h, 0)),
            scratch_shapes=[pltpu.VMEM((kW * Cin, XW), jnp.bfloat16)],
        ),
        compiler_params=pltpu.CompilerParams(
            dimension_semantics=("parallel", "parallel")),
    )(x_nchw, halo, ml, mr, w_k, b2)

    return out


# speedup vs baseline: 9.5813x; 2.1731x over previous
import jax
import jax.numpy as jnp
from jax.experimental import pallas as pl
from jax.experimental.pallas import tpu as pltpu


def _conv3x3_kernel(x_ref, ml_ref, mr_ref, w_ref, b_ref, o_ref, pk_ref):
    C = x_ref.shape[1]
    H = x_ref.shape[2]
    W = x_ref.shape[3]
    S = H * W
    Cout = o_ref.shape[1]

    zpad = jnp.zeros((3 * C, W), jnp.bfloat16)
    pk_ref[:, 0:W] = zpad
    pk_ref[:, W + S:] = zpad
    pk_ref[C:2 * C, W:W + S] = x_ref[0].astype(jnp.bfloat16).reshape(C, S)

    cen = pk_ref[C:2 * C, :]
    z1 = jnp.zeros((C, 1), jnp.bfloat16)
    pk_ref[0:C, :] = jnp.concatenate([z1, cen[:, :-1]], axis=1) * ml_ref[...]
    pk_ref[2 * C:3 * C, :] = (jnp.concatenate([cen[:, 1:], z1], axis=1)
                              * mr_ref[...])

    acc = jnp.broadcast_to(b_ref[...], (Cout, S)).astype(jnp.float32)
    for kh in range(3):
        acc = acc + jnp.dot(w_ref[kh], pk_ref[:, kh * W:kh * W + S],
                            preferred_element_type=jnp.float32)

    o_ref[0] = jnp.maximum(acc, 0.0).reshape(Cout, H, W)


def kernel(x_nchw, w_hwio, bias):
    N, Cin, H, W = x_nchw.shape
    kH, kW, _, Cout = w_hwio.shape
    assert (kH, kW) == (3, 3)

    S = H * W
    XW = S + 2 * W

    lane = jnp.arange(XW, dtype=jnp.int32) % W
    ml = (lane != 0).astype(jnp.bfloat16).reshape(1, XW)
    mr = (lane != W - 1).astype(jnp.bfloat16).reshape(1, XW)

    w_k = jnp.transpose(w_hwio, (0, 3, 1, 2)).reshape(kH, Cout, kW * Cin)
    w_k = w_k.astype(jnp.bfloat16)
    b2 = bias.reshape(Cout, 1).astype(jnp.float32)

    out = pl.pallas_call(
        _conv3x3_kernel,
        out_shape=jax.ShapeDtypeStruct((N, Cout, H, W), jnp.float32),
        grid_spec=pltpu.PrefetchScalarGridSpec(
            num_scalar_prefetch=0,
            grid=(N,),
            in_specs=[
                pl.BlockSpec((1, Cin, H, W), lambda n: (n, 0, 0, 0)),
                pl.BlockSpec((1, XW), lambda n: (0, 0)),
                pl.BlockSpec((1, XW), lambda n: (0, 0)),
                pl.BlockSpec((kH, Cout, kW * Cin), lambda n: (0, 0, 0)),
                pl.BlockSpec((Cout, 1), lambda n: (0, 0)),
            ],
            out_specs=pl.BlockSpec((1, Cout, H, W), lambda n: (n, 0, 0, 0)),
            scratch_shapes=[pltpu.VMEM((kW * Cin, XW), jnp.bfloat16)],
        ),
        compiler_params=pltpu.CompilerParams(
            dimension_semantics=("parallel",)),
    )(x_nchw, ml, mr, w_k, b2)

    return out
